# Initial kernel scaffold; baseline (speedup 1.0000x reference)
import jax
import jax.numpy as jnp
from jax.experimental import pallas as pl


def kernel(x, edge_index, W1, b1, W2, b2):
    raise NotImplementedError("write your pallas kernel here")


# speedup vs baseline: 13.9387x; 13.9387x over previous
import functools

import jax
import jax.numpy as jnp
from jax import lax
from jax.experimental import pallas as pl
from jax.experimental.pallas import tpu as pltpu
from jax.experimental.pallas import tpu_sc as plsc

NC = 2
NS = 16
K = 128


def _mesh():
    return plsc.VectorSubcoreMesh(core_axis_name="c", subcore_axis_name="s")


def _make_deg_kernel(N, E, D):
    EPC = E // NC
    EPW = EPC // NS
    NCH = EPW // K
    TAIL = EPW % K
    TT = TAIL if TAIL else 8
    NPT = (pl.cdiv(N, NS) + 7) // 8 * 8
    NP = NPT * NS

    @functools.partial(
        pl.kernel,
        out_type=jax.ShapeDtypeStruct((NC, NP, D), jnp.float32),
        mesh=_mesh(),
        scratch_types=[
            pltpu.VMEM_SHARED((NP, D), jnp.float32),
            pltpu.VMEM((K,), jnp.int32),
            pltpu.VMEM((TT,), jnp.int32),
            pltpu.VMEM((K, D), jnp.float32),
            pltpu.VMEM((TT, D), jnp.float32),
        ],
    )
    def deg_kernel(dst_hbm, ones_hbm, zeros_hbm, out_hbm, degsp, idx, idx_t,
                   ones_v, ones_t):
        c = lax.axis_index("c")
        s = lax.axis_index("s")
        pltpu.sync_copy(zeros_hbm, degsp.at[pl.ds(s * NPT, NPT)])
        pltpu.sync_copy(ones_hbm, ones_v)
        pltpu.sync_copy(ones_hbm.at[pl.ds(0, TT)], ones_t)
        plsc.subcore_barrier()
        base = c * EPC + s * EPW

        def chunk(g, carry):
            b = base + g * K
            pltpu.sync_copy(dst_hbm.at[pl.ds(b, K)], idx)
            pltpu.sync_copy(ones_v, degsp.at[idx], add=True)
            return carry

        lax.fori_loop(0, NCH, chunk, 0)
        if TAIL:
            b = base + NCH * K
            pltpu.sync_copy(dst_hbm.at[pl.ds(b, TAIL)], idx_t)
            pltpu.sync_copy(ones_t, degsp.at[idx_t], add=True)
        plsc.subcore_barrier()
        pltpu.sync_copy(degsp.at[pl.ds(s * NPT, NPT)],
                        out_hbm.at[c, pl.ds(s * NPT, NPT)])

    return deg_kernel


def _make_edge_kernel(N, E, D):
    EPC = E // NC
    EPW = EPC // NS
    NCH = EPW // K
    TAIL = EPW % K
    TT = TAIL if TAIL else 8
    NPT = (pl.cdiv(N, NS) + 7) // 8 * 8
    NP = NPT * NS

    @functools.partial(
        pl.kernel,
        out_type=jax.ShapeDtypeStruct((NC, NP, D), jnp.float32),
        mesh=_mesh(),
        scratch_types=[
            pltpu.VMEM_SHARED((NP, D), jnp.float32),
            pltpu.VMEM((K,), jnp.int32),
            pltpu.VMEM((K,), jnp.int32),
            pltpu.VMEM((K, D), jnp.float32),
            pltpu.VMEM((TT,), jnp.int32),
            pltpu.VMEM((TT,), jnp.int32),
            pltpu.VMEM((TT, D), jnp.float32),
            pltpu.SemaphoreType.DMA,
        ],
    )
    def edge_kernel(hs_hbm, src_hbm, dst_hbm, zeros_hbm, out_hbm, accsp,
                    idxs, idxd, rows, idxs_t, idxd_t, rows_t, sem):
        c = lax.axis_index("c")
        s = lax.axis_index("s")
        pltpu.sync_copy(zeros_hbm, accsp.at[pl.ds(s * NPT, NPT)])
        plsc.subcore_barrier()
        base = c * EPC + s * EPW

        def chunk(g, carry):
            b = base + g * K
            pltpu.sync_copy(src_hbm.at[pl.ds(b, K)], idxs)
            pltpu.sync_copy(dst_hbm.at[pl.ds(b, K)], idxd)
            pltpu.async_copy(hs_hbm.at[idxs], rows, sem).wait()
            pltpu.sync_copy(rows, accsp.at[idxd], add=True)
            return carry

        lax.fori_loop(0, NCH, chunk, 0)
        if TAIL:
            b = base + NCH * K
            pltpu.sync_copy(src_hbm.at[pl.ds(b, TAIL)], idxs_t)
            pltpu.sync_copy(dst_hbm.at[pl.ds(b, TAIL)], idxd_t)
            pltpu.async_copy(hs_hbm.at[idxs_t], rows_t, sem).wait()
            pltpu.sync_copy(rows_t, accsp.at[idxd_t], add=True)
        plsc.subcore_barrier()
        pltpu.sync_copy(accsp.at[pl.ds(s * NPT, NPT)],
                        out_hbm.at[c, pl.ds(s * NPT, NPT)])

    return edge_kernel


def _dis_from_deg(degp_ref):
    deg = degp_ref[0][:, 0:1] + degp_ref[1][:, 0:1] + 1.0
    return lax.rsqrt(deg)


def _tc_first(x_ref, w_ref, degp_ref, hs_ref):
    dis = _dis_from_deg(degp_ref)
    h = jnp.dot(x_ref[...], w_ref[...], preferred_element_type=jnp.float32,
                precision=lax.Precision.HIGHEST)
    hs_ref[...] = h * dis


def _tc_mid(acc_ref, hs_ref, degp_ref, b_ref, w_ref, out_ref):
    dis = _dis_from_deg(degp_ref)
    t = dis * (acc_ref[0] + acc_ref[1] + hs_ref[...]) + b_ref[...]
    o1 = jnp.maximum(t, 0.0)
    h2 = jnp.dot(o1, w_ref[...], preferred_element_type=jnp.float32,
                 precision=lax.Precision.HIGHEST)
    out_ref[...] = h2 * dis


def _tc_last(acc_ref, hs_ref, degp_ref, b_ref, out_ref):
    dis = _dis_from_deg(degp_ref)
    out_ref[...] = dis * (acc_ref[0] + acc_ref[1] + hs_ref[...]) + b_ref[...]


def kernel(x, edge_index, W1, b1, W2, b2):
    N, D = x.shape
    E = edge_index.shape[1]
    assert E % (NC * NS) == 0 and (E // (NC * NS)) % 8 == 0
    assert N % NS == 0

    src = edge_index[0]
    dst = edge_index[1]
    f32 = jnp.float32
    NPT = (pl.cdiv(N, NS) + 7) // 8 * 8
    onesD = jnp.ones((K, D), f32)
    zerosD = jnp.zeros((NPT, D), f32)

    degp = _make_deg_kernel(N, E, D)(dst, onesD, zerosD)[:, :N]

    BN = 400
    grid = (N // BN,)
    blk_nd = pl.BlockSpec((BN, D), lambda i: (i, 0))
    blk_w = pl.BlockSpec((D, D), lambda i: (0, 0))
    blk_deg = pl.BlockSpec((NC, BN, D), lambda i: (0, i, 0))
    blk_acc = pl.BlockSpec((NC, BN, D), lambda i: (0, i, 0))
    blk_b = pl.BlockSpec((1, D), lambda i: (0, 0))
    out_nd = jax.ShapeDtypeStruct((N, D), f32)

    hs1 = pl.pallas_call(
        _tc_first, grid=grid,
        in_specs=[blk_nd, blk_w, blk_deg],
        out_specs=blk_nd, out_shape=out_nd,
    )(x, W1, degp)

    edge_k = _make_edge_kernel(N, E, D)
    acc1 = edge_k(hs1, src, dst, zerosD)[:, :N]

    hs2 = pl.pallas_call(
        _tc_mid, grid=grid,
        in_specs=[blk_acc, blk_nd, blk_deg, blk_b, blk_w],
        out_specs=blk_nd, out_shape=out_nd,
    )(acc1, hs1, degp, b1.reshape(1, D), W2)

    acc2 = edge_k(hs2, src, dst, zerosD)[:, :N]

    out = pl.pallas_call(
        _tc_last, grid=grid,
        in_specs=[blk_acc, blk_nd, blk_deg, blk_b],
        out_specs=blk_nd, out_shape=out_nd,
    )(acc2, hs2, degp, b2.reshape(1, D))
    return out
